# Initial kernel scaffold; baseline (speedup 1.0000x reference)
import functools

import jax
import jax.numpy as jnp
from jax import lax
from jax.experimental import pallas as pl
from jax.experimental.pallas import tpu as pltpu
from jax.experimental.pallas import tpu_sc as plsc

N = 50000
E = 800000
H = 64
NG = 64
NC = 10

NT = 16
NSC = 2
NPAD = 50048
BN = NPAD // NT

CW = 80
EROWS = E // CW
RPT = EROWS // NT
BLK = 25
NBLK = RPT // BLK

_mesh = plsc.VectorSubcoreMesh(core_axis_name="c", subcore_axis_name="s")


@functools.partial(
    pl.kernel,
    out_type=(
        jax.ShapeDtypeStruct((NSC, NPAD), jnp.float32),
        jax.ShapeDtypeStruct((NSC, NPAD), jnp.float32),
    ),
    mesh=_mesh,
    scratch_types=[
        pltpu.VMEM((BLK, CW), jnp.int32),
        pltpu.VMEM((BLK, CW), jnp.int32),
        pltpu.VMEM((CW,), jnp.float32),
        pltpu.VMEM_SHARED((NPAD,), jnp.float32),
        pltpu.VMEM_SHARED((NPAD,), jnp.float32),
    ],
)
def _deg_kernel(srcs_hbm, dsts_hbm, z1_hbm, od_hbm, id_hbm,
                src_blk, dst_blk, ones_v, od_sh, id_sh):
    c = lax.axis_index("c")
    t = lax.axis_index("s")

    for i in range(CW // 16):
        ones_v[pl.ds(i * 16, 16)] = jnp.full((16,), 1.0, jnp.float32)

    pltpu.sync_copy(z1_hbm, od_sh.at[pl.ds(t * BN, BN)])
    pltpu.sync_copy(z1_hbm, id_sh.at[pl.ds(t * BN, BN)])
    plsc.subcore_barrier()

    def block_body(blk, _):
        row = t * RPT + blk * BLK
        pltpu.sync_copy(srcs_hbm.at[c, pl.ds(row, BLK), :], src_blk)
        pltpu.sync_copy(dsts_hbm.at[c, pl.ds(row, BLK), :], dst_blk)
        for j in range(BLK):
            pltpu.sync_copy(ones_v, od_sh.at[src_blk.at[j]], add=True)
            pltpu.sync_copy(ones_v, id_sh.at[dst_blk.at[j]], add=True)
        return _

    lax.fori_loop(0, NBLK, block_body, 0)
    plsc.subcore_barrier()

    pltpu.sync_copy(od_sh.at[pl.ds(t * BN, BN)], od_hbm.at[c, pl.ds(t * BN, BN)])
    pltpu.sync_copy(id_sh.at[pl.ds(t * BN, BN)], id_hbm.at[c, pl.ds(t * BN, BN)])


@functools.partial(
    pl.kernel,
    out_type=jax.ShapeDtypeStruct((NSC, NPAD, 32), jnp.float32),
    mesh=_mesh,
    scratch_types=[
        pltpu.VMEM((BLK, CW), jnp.int32),
        pltpu.VMEM((BLK, CW), jnp.int32),
        pltpu.VMEM((CW, 32), jnp.float32),
        pltpu.VMEM((CW, 32), jnp.float32),
        pltpu.VMEM_SHARED((NPAD, 32), jnp.float32),
        pltpu.SemaphoreType.DMA,
    ],
)
def _rowpass_kernel(srcoff_hbm, dst_hbm, ms_hbm, zrows_hbm, agg_hbm,
                    src_blk, dst_blk, rows_a, rows_b, acc_sh, gsem):
    c = lax.axis_index("c")
    t = lax.axis_index("s")

    pltpu.sync_copy(zrows_hbm, acc_sh.at[pl.ds(t * BN, BN), :])
    plsc.subcore_barrier()

    bufs = (rows_a, rows_b)

    def block_body(blk, _):
        row = t * RPT + blk * BLK
        pltpu.sync_copy(srcoff_hbm.at[c, pl.ds(row, BLK), :], src_blk)
        pltpu.sync_copy(dst_hbm.at[pl.ds(row, BLK), :], dst_blk)
        pltpu.async_copy(ms_hbm.at[src_blk.at[0]], bufs[0], gsem).wait()
        for j in range(BLK):
            if j + 1 < BLK:
                nxt = pltpu.async_copy(
                    ms_hbm.at[src_blk.at[j + 1]], bufs[(j + 1) % 2], gsem)
            pltpu.sync_copy(bufs[j % 2], acc_sh.at[dst_blk.at[j]], add=True)
            if j + 1 < BLK:
                nxt.wait()
        return _

    lax.fori_loop(0, NBLK, block_body, 0)
    plsc.subcore_barrier()

    pltpu.sync_copy(acc_sh.at[pl.ds(t * BN, BN), :],
                    agg_hbm.at[c, pl.ds(t * BN, BN), :])


def _prep_body(od_ref, id_ref, w1_ref, no_ref, ni_ref, m_ref):
    od = od_ref[...]
    idg = id_ref[...]
    no = lax.rsqrt(jnp.maximum(od, 1.0))
    ni = lax.rsqrt(jnp.maximum(idg, 1.0))
    no_ref[...] = no
    ni_ref[...] = ni
    m = (idg * no) * w1_ref[...]
    m_ref[0, :, :] = m[:, :32]
    m_ref[1, :, :] = m[:, 32:]


_prep_call = pl.pallas_call(
    _prep_body,
    grid=(NT,),
    in_specs=[
        pl.BlockSpec((BN, 1), lambda i: (i, 0)),
        pl.BlockSpec((BN, 1), lambda i: (i, 0)),
        pl.BlockSpec((1, H), lambda i: (0, 0)),
    ],
    out_specs=[
        pl.BlockSpec((BN, 1), lambda i: (i, 0)),
        pl.BlockSpec((BN, 1), lambda i: (i, 0)),
        pl.BlockSpec((NSC, BN, 32), lambda i: (0, i, 0)),
    ],
    out_shape=[
        jax.ShapeDtypeStruct((NPAD, 1), jnp.float32),
        jax.ShapeDtypeStruct((NPAD, 1), jnp.float32),
        jax.ShapeDtypeStruct((NSC, NPAD, 32), jnp.float32),
    ],
)


def _layer_body(agg_ref, ni_ref, no_ref, b_ref, w_ref, m_ref):
    aggf = jnp.concatenate([agg_ref[0, :, :], agg_ref[1, :, :]], axis=1)
    h = jax.nn.relu(aggf * ni_ref[...] + b_ref[...])
    m = jnp.dot(h * no_ref[...], w_ref[...], preferred_element_type=jnp.float32)
    m_ref[0, :, :] = m[:, :32]
    m_ref[1, :, :] = m[:, 32:]


_layer_call = pl.pallas_call(
    _layer_body,
    grid=(NT,),
    in_specs=[
        pl.BlockSpec((NSC, BN, 32), lambda i: (0, i, 0)),
        pl.BlockSpec((BN, 1), lambda i: (i, 0)),
        pl.BlockSpec((BN, 1), lambda i: (i, 0)),
        pl.BlockSpec((1, H), lambda i: (0, 0)),
        pl.BlockSpec((H, H), lambda i: (0, 0)),
    ],
    out_specs=pl.BlockSpec((NSC, BN, 32), lambda i: (0, i, 0)),
    out_shape=jax.ShapeDtypeStruct((NSC, NPAD, 32), jnp.float32),
)


def _pool_body(agg_ref, ni_ref, b_ref, gid_ref, sums_ref, counts_ref):
    i = pl.program_id(0)
    aggf = jnp.concatenate([agg_ref[0, :, :], agg_ref[1, :, :]], axis=1)
    h = jax.nn.relu(aggf * ni_ref[...] + b_ref[...])
    gid = gid_ref[...]
    cols = lax.broadcasted_iota(jnp.int32, (BN, NG), 1)
    g = (gid == cols).astype(jnp.float32)
    sums = lax.dot_general(g, h, (((0,), (0,)), ((), ())),
                           preferred_element_type=jnp.float32)
    counts = lax.dot_general(g, jnp.ones((BN, 1), jnp.float32),
                             (((0,), (0,)), ((), ())),
                             preferred_element_type=jnp.float32)

    @pl.when(i == 0)
    def _():
        sums_ref[...] = jnp.zeros_like(sums_ref)
        counts_ref[...] = jnp.zeros_like(counts_ref)

    sums_ref[...] += sums
    counts_ref[...] += counts


_pool_call = pl.pallas_call(
    _pool_body,
    grid=(NT,),
    in_specs=[
        pl.BlockSpec((NSC, BN, 32), lambda i: (0, i, 0)),
        pl.BlockSpec((BN, 1), lambda i: (i, 0)),
        pl.BlockSpec((1, H), lambda i: (0, 0)),
        pl.BlockSpec((BN, 1), lambda i: (i, 0)),
    ],
    out_specs=[
        pl.BlockSpec((NG, H), lambda i: (0, 0)),
        pl.BlockSpec((NG, 1), lambda i: (0, 0)),
    ],
    out_shape=[
        jax.ShapeDtypeStruct((NG, H), jnp.float32),
        jax.ShapeDtypeStruct((NG, 1), jnp.float32),
    ],
)


def _combine_body(s1_ref, c1_ref, s2_ref, c2_ref, wc_ref, bc_ref,
                  hg1_ref, hg2_ref, logits_ref):
    hg1 = s1_ref[...] / jnp.maximum(c1_ref[...], 1.0)
    hg2 = s2_ref[...] / jnp.maximum(c2_ref[...], 1.0)
    hg1_ref[...] = hg1
    hg2_ref[...] = hg2
    hg = jnp.abs(hg1 - hg2)
    logits_ref[...] = (
        jnp.dot(hg, wc_ref[...], preferred_element_type=jnp.float32)
        + bc_ref[...])


_combine_call = pl.pallas_call(
    _combine_body,
    out_shape=[
        jax.ShapeDtypeStruct((NG, H), jnp.float32),
        jax.ShapeDtypeStruct((NG, H), jnp.float32),
        jax.ShapeDtypeStruct((NG, NC), jnp.float32),
    ],
)


def _branch_outputs(srcoff, dst_r, od, idg, gid, W1, b1, W2, b2, W3, b3, W4, b4,
                    zrows):
    od2 = od.reshape(NPAD, 1)
    id2 = idg.reshape(NPAD, 1)
    no, ni, m1 = _prep_call(od2, id2, W1)
    weights = [(b1, W2), (b2, W3), (b3, W4)]
    m = m1
    for b, wn in weights:
        agg = _rowpass_kernel(srcoff, dst_r, m.reshape(NSC * NPAD, 32), zrows)
        m = _layer_call(agg, ni, no, b.reshape(1, H), wn)
    agg4 = _rowpass_kernel(srcoff, dst_r, m.reshape(NSC * NPAD, 32), zrows)
    sums, counts = _pool_call(agg4, ni, b4.reshape(1, H), gid)
    return sums, counts


def kernel(edge_index1, edge_index2, graph_ids1, graph_ids2,
           W1, b1, W2, b2, W3, b3, W4, b4, Wc, bc):
    src1 = edge_index1[0].reshape(EROWS, CW)
    dst1 = edge_index1[1].reshape(EROWS, CW)
    src2 = edge_index2[0].reshape(EROWS, CW)
    dst2 = edge_index2[1].reshape(EROWS, CW)

    srcs = jnp.stack([src1, src2])
    dsts = jnp.stack([dst1, dst2])
    z1 = jnp.zeros((BN,), jnp.float32)
    zrows = jnp.zeros((BN, 32), jnp.float32)

    od, idg = _deg_kernel(srcs, dsts, z1)

    srcoff1 = jnp.stack([src1, src1 + NPAD])
    srcoff2 = jnp.stack([src2, src2 + NPAD])

    pad = jnp.full((NPAD - N, 1), NG, jnp.int32)
    gid1 = jnp.concatenate([graph_ids1.reshape(N, 1), pad], axis=0)
    gid2 = jnp.concatenate([graph_ids2.reshape(N, 1), pad], axis=0)

    s1, c1 = _branch_outputs(srcoff1, dst1, od[0], idg[0], gid1,
                             W1, b1, W2, b2, W3, b3, W4, b4, zrows)
    s2, c2 = _branch_outputs(srcoff2, dst2, od[1], idg[1], gid2,
                             W1, b1, W2, b2, W3, b3, W4, b4, zrows)

    return _combine_call(s1, c1, s2, c2, Wc, bc.reshape(1, NC))


# speedup vs baseline: 2.5910x; 2.5910x over previous
import functools

import jax
import jax.numpy as jnp
from jax import lax
from jax.experimental import pallas as pl
from jax.experimental.pallas import tpu as pltpu
from jax.experimental.pallas import tpu_sc as plsc

N = 50000
E = 800000
H = 64
NG = 64
NC = 10

NT = 16
NSC = 2
NQ = 4
QW = H // NQ
NPAD = 51200
BN = NPAD // NT

CW = 128
BLK = 40
NBLK = 10
RPT = BLK * NBLK
ERP = NT * RPT
EPAD = ERP * CW - E

ZR = 800

_mesh = plsc.VectorSubcoreMesh(core_axis_name="c", subcore_axis_name="s")
_sc_params = pltpu.CompilerParams(use_tc_tiling_on_sc=False)


@functools.partial(
    pl.kernel,
    out_type=(
        jax.ShapeDtypeStruct((NSC, NPAD), jnp.float32),
        jax.ShapeDtypeStruct((NSC, NPAD), jnp.float32),
    ),
    mesh=_mesh,
    compiler_params=_sc_params,
    scratch_types=[
        pltpu.VMEM((BLK, 1, CW), jnp.int32),
        pltpu.VMEM((BLK, 1, CW), jnp.int32),
        pltpu.VMEM((CW,), jnp.float32),
        pltpu.VMEM((BN,), jnp.float32),
        pltpu.VMEM_SHARED((NPAD,), jnp.float32),
        pltpu.VMEM_SHARED((NPAD,), jnp.float32),
    ],
)
def _deg_kernel(srcs_hbm, dsts_hbm, od_hbm, id_hbm,
                src_blk, dst_blk, ones_v, zb, od_sh, id_sh):
    c = lax.axis_index("c")
    t = lax.axis_index("s")

    one = jnp.full((16,), 1.0, jnp.float32)
    z = jnp.zeros((16,), jnp.float32)
    for i in range(CW // 16):
        ones_v[pl.ds(i * 16, 16)] = one
    for i in range(BN // 16):
        zb[pl.ds(i * 16, 16)] = z

    pltpu.sync_copy(zb, od_sh.at[pl.ds(t * BN, BN)])
    pltpu.sync_copy(zb, id_sh.at[pl.ds(t * BN, BN)])
    plsc.subcore_barrier()

    def block_body(blk, carry):
        row = t * RPT + blk * BLK
        pltpu.sync_copy(srcs_hbm.at[c, pl.ds(row, BLK), :, :], src_blk)
        pltpu.sync_copy(dsts_hbm.at[c, pl.ds(row, BLK), :, :], dst_blk)
        for j in range(BLK):
            pltpu.sync_copy(ones_v, od_sh.at[src_blk.at[j, 0]], add=True)
            pltpu.sync_copy(ones_v, id_sh.at[dst_blk.at[j, 0]], add=True)
        return carry

    lax.fori_loop(0, NBLK, block_body, 0)
    plsc.subcore_barrier()

    pltpu.sync_copy(od_sh.at[pl.ds(t * BN, BN)], od_hbm.at[c, pl.ds(t * BN, BN)])
    pltpu.sync_copy(id_sh.at[pl.ds(t * BN, BN)], id_hbm.at[c, pl.ds(t * BN, BN)])


@functools.partial(
    pl.kernel,
    out_type=(
        jax.ShapeDtypeStruct((NPAD, 128), jnp.float32),
        jax.ShapeDtypeStruct((NQ * NPAD, QW), jnp.float32),
    ),
    mesh=_mesh,
    compiler_params=_sc_params,
    scratch_types=[
        pltpu.VMEM((BLK, 1, CW), jnp.int32),
        pltpu.VMEM((BLK, 1, CW), jnp.int32),
        pltpu.VMEM((CW, QW), jnp.float32),
        pltpu.VMEM((CW, QW), jnp.float32),
        pltpu.VMEM((ZR, QW), jnp.float32),
        pltpu.VMEM((BN, QW), jnp.float32),
        pltpu.VMEM_SHARED((NPAD, QW), jnp.float32),
        pltpu.SemaphoreType.DMA,
    ],
)
def _rowpass_kernel(srcq_hbm, dst_hbm, m_hbm, agg_hbm, msqt_hbm,
                    src_blk, dst_blk, rows_a, rows_b, zb, st_buf, acc_sh, gsem):
    c = lax.axis_index("c")
    t = lax.axis_index("s")

    z = jnp.zeros((16,), jnp.float32)
    for i in range(ZR):
        zb[i, pl.ds(0, 16)] = z

    for p in range(2):
        q = 2 * c + p
        pltpu.sync_copy(m_hbm.at[pl.ds(t * BN, BN), pl.ds(q * QW, QW)], st_buf)
        pltpu.sync_copy(st_buf, msqt_hbm.at[pl.ds(q * NPAD + t * BN, BN), :])

    bufs = (rows_a, rows_b)

    for p in range(2):
        q = 2 * c + p

        for k in range(BN // ZR):
            pltpu.sync_copy(zb, acc_sh.at[pl.ds(t * BN + k * ZR, ZR), :])
        plsc.subcore_barrier()

        def block_body(blk, carry):
            row = t * RPT + blk * BLK
            pltpu.sync_copy(srcq_hbm.at[q, pl.ds(row, BLK), :, :], src_blk)
            pltpu.sync_copy(dst_hbm.at[pl.ds(row, BLK), :, :], dst_blk)
            pltpu.async_copy(msqt_hbm.at[src_blk.at[0, 0]], bufs[0], gsem).wait()
            for j in range(BLK):
                if j + 1 < BLK:
                    nxt = pltpu.async_copy(
                        msqt_hbm.at[src_blk.at[j + 1, 0]], bufs[(j + 1) % 2], gsem)
                pltpu.sync_copy(bufs[j % 2], acc_sh.at[dst_blk.at[j, 0]], add=True)
                if j + 1 < BLK:
                    nxt.wait()
            return carry

        lax.fori_loop(0, NBLK, block_body, 0)
        plsc.subcore_barrier()

        pltpu.sync_copy(acc_sh.at[pl.ds(t * BN, BN), :],
                        agg_hbm.at[pl.ds(t * BN, BN), pl.ds(q * QW, QW)])


def _expand_grid(grid_vals):
    gb = grid_vals.shape[0]
    rep = jnp.broadcast_to(grid_vals[:, None, :], (gb, 128, 128))
    rep = rep.reshape(gb * 128, 128)
    lane = lax.broadcasted_iota(jnp.int32, (gb * 128, 128), 1)
    rowmod = lax.broadcasted_iota(jnp.int32, (gb * 128, 128), 0) % 128
    return jnp.where(lane == rowmod, rep, 0.0)


def _to_dense(grid_vals, k):
    a = _expand_grid(grid_vals)
    return jnp.dot(a, jnp.ones((128, k), jnp.float32),
                   preferred_element_type=jnp.float32,
                   precision=lax.Precision.HIGHEST)


def _prep_body(od_ref, id_ref, w1_ref, no_ref, ni_ref, m_ref):
    no_g = lax.rsqrt(jnp.maximum(od_ref[0], 1.0))
    ni_g = lax.rsqrt(jnp.maximum(id_ref[0], 1.0))
    zpad = jnp.zeros((BN, 128 - H), jnp.float32)
    no_ref[:, :H] = _to_dense(no_g, H)
    no_ref[:, H:] = zpad
    ni_ref[:, :H] = _to_dense(ni_g, H)
    ni_ref[:, H:] = zpad
    w1b = jnp.broadcast_to(w1_ref[...], (128, H))
    m1 = jnp.dot(_expand_grid(id_ref[0] * no_g), w1b,
                 preferred_element_type=jnp.float32,
                   precision=lax.Precision.HIGHEST)
    m_ref[:, :H] = m1
    m_ref[:, H:] = zpad


_prep_call = pl.pallas_call(
    _prep_body,
    grid=(NT,),
    in_specs=[
        pl.BlockSpec((1, BN // 128, 128), lambda i: (i, 0, 0)),
        pl.BlockSpec((1, BN // 128, 128), lambda i: (i, 0, 0)),
        pl.BlockSpec((1, H), lambda i: (0, 0)),
    ],
    out_specs=[
        pl.BlockSpec((BN, 128), lambda i: (i, 0)),
        pl.BlockSpec((BN, 128), lambda i: (i, 0)),
        pl.BlockSpec((BN, 128), lambda i: (i, 0)),
    ],
    out_shape=[
        jax.ShapeDtypeStruct((NPAD, 128), jnp.float32),
        jax.ShapeDtypeStruct((NPAD, 128), jnp.float32),
        jax.ShapeDtypeStruct((NPAD, 128), jnp.float32),
    ],
)


def _layer_body(agg_ref, ni_ref, no_ref, b_ref, w_ref, m_ref):
    aggf = agg_ref[:, :H]
    h = jax.nn.relu(aggf * ni_ref[:, :H] + b_ref[...])
    m = jnp.dot(h * no_ref[:, :H], w_ref[...], preferred_element_type=jnp.float32,
                   precision=lax.Precision.HIGHEST)
    m_ref[:, :H] = m
    m_ref[:, H:] = jnp.zeros((BN, 128 - H), jnp.float32)


_layer_call = pl.pallas_call(
    _layer_body,
    grid=(NT,),
    in_specs=[
        pl.BlockSpec((BN, 128), lambda i: (i, 0)),
        pl.BlockSpec((BN, 128), lambda i: (i, 0)),
        pl.BlockSpec((BN, 128), lambda i: (i, 0)),
        pl.BlockSpec((1, H), lambda i: (0, 0)),
        pl.BlockSpec((H, H), lambda i: (0, 0)),
    ],
    out_specs=pl.BlockSpec((BN, 128), lambda i: (i, 0)),
    out_shape=jax.ShapeDtypeStruct((NPAD, 128), jnp.float32),
)


def _pool_body(agg_ref, ni_ref, b_ref, gid_ref, sums_ref, counts_ref):
    i = pl.program_id(0)
    aggf = agg_ref[:, :H]
    h = jax.nn.relu(aggf * ni_ref[:, :H] + b_ref[...])
    gidcol = jnp.dot(_expand_grid(gid_ref[0].astype(jnp.float32)),
                     jnp.ones((128, 1), jnp.float32),
                     preferred_element_type=jnp.float32,
                   precision=lax.Precision.HIGHEST)
    cols = lax.broadcasted_iota(jnp.int32, (BN, NG), 1).astype(jnp.float32)
    g = (gidcol == cols).astype(jnp.float32)
    sums = lax.dot_general(g, h, (((0,), (0,)), ((), ())),
                           preferred_element_type=jnp.float32,
                   precision=lax.Precision.HIGHEST)
    counts = lax.dot_general(g, jnp.ones((BN, 1), jnp.float32),
                             (((0,), (0,)), ((), ())),
                             preferred_element_type=jnp.float32,
                   precision=lax.Precision.HIGHEST)

    @pl.when(i == 0)
    def _():
        sums_ref[...] = jnp.zeros_like(sums_ref)
        counts_ref[...] = jnp.zeros_like(counts_ref)

    sums_ref[...] += sums
    counts_ref[...] += counts


_pool_call = pl.pallas_call(
    _pool_body,
    grid=(NT,),
    in_specs=[
        pl.BlockSpec((BN, 128), lambda i: (i, 0)),
        pl.BlockSpec((BN, 128), lambda i: (i, 0)),
        pl.BlockSpec((1, H), lambda i: (0, 0)),
        pl.BlockSpec((1, BN // 128, 128), lambda i: (i, 0, 0)),
    ],
    out_specs=[
        pl.BlockSpec((NG, H), lambda i: (0, 0)),
        pl.BlockSpec((NG, 1), lambda i: (0, 0)),
    ],
    out_shape=[
        jax.ShapeDtypeStruct((NG, H), jnp.float32),
        jax.ShapeDtypeStruct((NG, 1), jnp.float32),
    ],
)


def _combine_body(s1_ref, c1_ref, s2_ref, c2_ref, wc_ref, bc_ref,
                  hg1_ref, hg2_ref, logits_ref):
    hg1 = s1_ref[...] / jnp.maximum(c1_ref[...], 1.0)
    hg2 = s2_ref[...] / jnp.maximum(c2_ref[...], 1.0)
    hg1_ref[...] = hg1
    hg2_ref[...] = hg2
    hg = jnp.abs(hg1 - hg2)
    logits_ref[...] = (
        jnp.dot(hg, wc_ref[...], preferred_element_type=jnp.float32,
                   precision=lax.Precision.HIGHEST)
        + bc_ref[...])


_combine_call = pl.pallas_call(
    _combine_body,
    out_shape=[
        jax.ShapeDtypeStruct((NG, H), jnp.float32),
        jax.ShapeDtypeStruct((NG, H), jnp.float32),
        jax.ShapeDtypeStruct((NG, NC), jnp.float32),
    ],
)


def _pad_edges(v, pad):
    return jnp.concatenate([v, pad]).reshape(ERP, 1, CW)


def _branch_outputs(srcq, dst_r, od, idg, gid, W1, b1, W2, b2, W3, b3, W4, b4):
    od_g = od.reshape(NT, BN // 128, 128)
    id_g = idg.reshape(NT, BN // 128, 128)
    no, ni, m1 = _prep_call(od_g, id_g, W1)
    m = m1
    for b, wn in [(b1, W2), (b2, W3), (b3, W4)]:
        agg, _ = _rowpass_kernel(srcq, dst_r, m)
        m = _layer_call(agg, ni, no, b.reshape(1, H), wn)
    agg4, _ = _rowpass_kernel(srcq, dst_r, m)
    sums, counts = _pool_call(agg4, ni, b4.reshape(1, H), gid)
    return sums, counts


def kernel(edge_index1, edge_index2, graph_ids1, graph_ids2,
           W1, b1, W2, b2, W3, b3, W4, b4, Wc, bc):
    pad = N + (jnp.arange(EPAD, dtype=jnp.int32) % (NPAD - N))
    src1 = _pad_edges(edge_index1[0], pad)
    dst1 = _pad_edges(edge_index1[1], pad)
    src2 = _pad_edges(edge_index2[0], pad)
    dst2 = _pad_edges(edge_index2[1], pad)

    srcs = jnp.stack([src1, src2])
    dsts = jnp.stack([dst1, dst2])

    od, idg = _deg_kernel(srcs, dsts)

    srcq1 = jnp.stack([src1 + q * NPAD for q in range(NQ)])
    srcq2 = jnp.stack([src2 + q * NPAD for q in range(NQ)])

    gpad = jnp.full((NPAD - N,), NG, jnp.int32)
    gid1 = jnp.concatenate([graph_ids1, gpad]).reshape(NT, BN // 128, 128)
    gid2 = jnp.concatenate([graph_ids2, gpad]).reshape(NT, BN // 128, 128)

    s1, c1 = _branch_outputs(srcq1, dst1, od[0], idg[0], gid1,
                             W1, b1, W2, b2, W3, b3, W4, b4)
    s2, c2 = _branch_outputs(srcq2, dst2, od[1], idg[1], gid2,
                             W1, b1, W2, b2, W3, b3, W4, b4)

    return _combine_call(s1, c1, s2, c2, Wc, bc.reshape(1, NC))


# speedup vs baseline: 5.2028x; 2.0081x over previous
import functools

import jax
import jax.numpy as jnp
from jax import lax
from jax.experimental import pallas as pl
from jax.experimental.pallas import tpu as pltpu
from jax.experimental.pallas import tpu_sc as plsc

N = 50000
E = 800000
H = 64
NG = 64
NC = 10

NT = 16
NSC = 2
NQ = 4
QW = H // NQ
NPAD = 51200
BN = NPAD // NT

CW = 128
BLK = 25
NBLK = 16
RPT = BLK * NBLK
ERP = NT * RPT
EPAD = ERP * CW - E

ZR = 400
GW = 13
SR = 400

_mesh = plsc.VectorSubcoreMesh(core_axis_name="c", subcore_axis_name="s")
_sc_params = pltpu.CompilerParams(use_tc_tiling_on_sc=False)


@functools.partial(
    pl.kernel,
    out_type=(
        jax.ShapeDtypeStruct((NSC, NPAD), jnp.float32),
        jax.ShapeDtypeStruct((NSC, NPAD), jnp.float32),
    ),
    mesh=_mesh,
    compiler_params=_sc_params,
    scratch_types=[
        pltpu.VMEM((BLK, 1, CW), jnp.int32),
        pltpu.VMEM((BLK, 1, CW), jnp.int32),
        pltpu.VMEM((CW,), jnp.float32),
        pltpu.VMEM((BN,), jnp.float32),
        pltpu.VMEM_SHARED((NPAD,), jnp.float32),
        pltpu.VMEM_SHARED((NPAD,), jnp.float32),
        pltpu.SemaphoreType.DMA,
    ],
)
def _deg_kernel(srcs_hbm, dsts_hbm, od_hbm, id_hbm,
                src_blk, dst_blk, ones_v, zb, od_sh, id_sh, dsem):
    c = lax.axis_index("c")
    t = lax.axis_index("s")

    one = jnp.full((16,), 1.0, jnp.float32)
    z = jnp.zeros((16,), jnp.float32)
    for i in range(CW // 16):
        ones_v[pl.ds(i * 16, 16)] = one
    for i in range(BN // 16):
        zb[pl.ds(i * 16, 16)] = z

    pltpu.sync_copy(zb, od_sh.at[pl.ds(t * BN, BN)])
    pltpu.sync_copy(zb, id_sh.at[pl.ds(t * BN, BN)])
    plsc.subcore_barrier()

    def block_body(blk, carry):
        row = t * RPT + blk * BLK
        pltpu.sync_copy(srcs_hbm.at[c, pl.ds(row, BLK), :, :], src_blk)
        pltpu.sync_copy(dsts_hbm.at[c, pl.ds(row, BLK), :, :], dst_blk)
        ds_ = []
        for j in range(BLK):
            ds_.append(pltpu.async_copy(
                ones_v, od_sh.at[src_blk.at[j, 0]], dsem, add=True))
            ds_.append(pltpu.async_copy(
                ones_v, id_sh.at[dst_blk.at[j, 0]], dsem, add=True))
        for d in ds_:
            d.wait()
        return carry

    lax.fori_loop(0, NBLK, block_body, 0)
    plsc.subcore_barrier()

    pltpu.sync_copy(od_sh.at[pl.ds(t * BN, BN)], od_hbm.at[c, pl.ds(t * BN, BN)])
    pltpu.sync_copy(id_sh.at[pl.ds(t * BN, BN)], id_hbm.at[c, pl.ds(t * BN, BN)])


@functools.partial(
    pl.kernel,
    out_type=(
        jax.ShapeDtypeStruct((NPAD, 128), jnp.float32),
        jax.ShapeDtypeStruct((NQ * NPAD, QW), jnp.float32),
    ),
    mesh=_mesh,
    compiler_params=_sc_params,
    scratch_types=[
        pltpu.VMEM((BLK, 1, CW), jnp.int32),
        pltpu.VMEM((BLK, 1, CW), jnp.int32),
        pltpu.VMEM((BLK, CW, QW), jnp.float32),
        pltpu.VMEM((ZR, QW), jnp.float32),
        pltpu.VMEM_SHARED((NPAD, QW), jnp.float32),
        pltpu.SemaphoreType.DMA,
        pltpu.SemaphoreType.DMA,
    ],
)
def _rowpass_kernel(srcq_hbm, dst_hbm, m_hbm, agg_hbm, msqt_hbm,
                    src_blk, dst_blk, ring, zb, acc_sh,
                    gsem, ssem):
    c = lax.axis_index("c")
    t = lax.axis_index("s")

    z = jnp.zeros((16,), jnp.float32)
    for i in range(ZR):
        zb[i, pl.ds(0, 16)] = z

    for p in range(2):
        q = 2 * c + p
        pltpu.sync_copy(m_hbm.at[pl.ds(t * BN, BN), pl.ds(q * QW, QW)],
                        acc_sh.at[pl.ds(t * BN, BN), :])
        pltpu.sync_copy(acc_sh.at[pl.ds(t * BN, BN), :],
                        msqt_hbm.at[pl.ds(q * NPAD + t * BN, BN), :])

    for p in range(2):
        q = 2 * c + p

        for k in range(BN // ZR):
            pltpu.sync_copy(zb, acc_sh.at[pl.ds(t * BN + k * ZR, ZR), :])
        plsc.subcore_barrier()

        def block_body(blk, carry):
            row = t * RPT + blk * BLK
            pltpu.sync_copy(srcq_hbm.at[q, pl.ds(row, BLK), :, :], src_blk)
            pltpu.sync_copy(dst_hbm.at[pl.ds(row, BLK), :, :], dst_blk)
            scats = []
            ga = [pltpu.async_copy(msqt_hbm.at[src_blk.at[j, 0]],
                                   ring.at[j], gsem)
                  for j in range(GW)]
            for j in range(GW):
                ga[j].wait()
            for j in range(GW):
                scats.append(pltpu.async_copy(
                    ring.at[j], acc_sh.at[dst_blk.at[j, 0]], ssem, add=True))
            gb = [pltpu.async_copy(msqt_hbm.at[src_blk.at[GW + j, 0]],
                                   ring.at[GW + j], gsem)
                  for j in range(BLK - GW)]
            for j in range(GW):
                scats[j].wait()
            for j in range(BLK - GW):
                gb[j].wait()
            for j in range(BLK - GW):
                scats.append(pltpu.async_copy(
                    ring.at[GW + j], acc_sh.at[dst_blk.at[GW + j, 0]],
                    ssem, add=True))
            for d in scats[GW:]:
                d.wait()
            return carry

        lax.fori_loop(0, NBLK, block_body, 0)
        plsc.subcore_barrier()

        pltpu.sync_copy(acc_sh.at[pl.ds(t * BN, BN), :],
                        agg_hbm.at[pl.ds(t * BN, BN), pl.ds(q * QW, QW)])


def _expand_grid(grid_vals):
    gb = grid_vals.shape[0]
    rep = jnp.broadcast_to(grid_vals[:, None, :], (gb, 128, 128))
    rep = rep.reshape(gb * 128, 128)
    lane = lax.broadcasted_iota(jnp.int32, (gb * 128, 128), 1)
    rowmod = lax.broadcasted_iota(jnp.int32, (gb * 128, 128), 0) % 128
    return jnp.where(lane == rowmod, rep, 0.0)


def _to_dense(grid_vals, k):
    a = _expand_grid(grid_vals)
    return jnp.dot(a, jnp.ones((128, k), jnp.float32),
                   preferred_element_type=jnp.float32,
                   precision=lax.Precision.HIGHEST)


def _prep_body(od_ref, id_ref, w1_ref, no_ref, ni_ref, m_ref):
    no_g = lax.rsqrt(jnp.maximum(od_ref[0], 1.0))
    ni_g = lax.rsqrt(jnp.maximum(id_ref[0], 1.0))
    zpad = jnp.zeros((BN, 128 - H), jnp.float32)
    no_ref[:, :H] = _to_dense(no_g, H)
    no_ref[:, H:] = zpad
    ni_ref[:, :H] = _to_dense(ni_g, H)
    ni_ref[:, H:] = zpad
    w1b = jnp.broadcast_to(w1_ref[...], (128, H))
    m1 = jnp.dot(_expand_grid(id_ref[0] * no_g), w1b,
                 preferred_element_type=jnp.float32,
                   precision=lax.Precision.HIGHEST)
    m_ref[:, :H] = m1
    m_ref[:, H:] = zpad


_prep_call = pl.pallas_call(
    _prep_body,
    grid=(NT,),
    in_specs=[
        pl.BlockSpec((1, BN // 128, 128), lambda i: (i, 0, 0)),
        pl.BlockSpec((1, BN // 128, 128), lambda i: (i, 0, 0)),
        pl.BlockSpec((1, H), lambda i: (0, 0)),
    ],
    out_specs=[
        pl.BlockSpec((BN, 128), lambda i: (i, 0)),
        pl.BlockSpec((BN, 128), lambda i: (i, 0)),
        pl.BlockSpec((BN, 128), lambda i: (i, 0)),
    ],
    out_shape=[
        jax.ShapeDtypeStruct((NPAD, 128), jnp.float32),
        jax.ShapeDtypeStruct((NPAD, 128), jnp.float32),
        jax.ShapeDtypeStruct((NPAD, 128), jnp.float32),
    ],
)


def _layer_body(agg_ref, ni_ref, no_ref, b_ref, w_ref, m_ref):
    aggf = agg_ref[:, :H]
    h = jax.nn.relu(aggf * ni_ref[:, :H] + b_ref[...])
    m = jnp.dot(h * no_ref[:, :H], w_ref[...], preferred_element_type=jnp.float32,
                   precision=lax.Precision.HIGHEST)
    m_ref[:, :H] = m
    m_ref[:, H:] = jnp.zeros((BN, 128 - H), jnp.float32)


_layer_call = pl.pallas_call(
    _layer_body,
    grid=(NT,),
    in_specs=[
        pl.BlockSpec((BN, 128), lambda i: (i, 0)),
        pl.BlockSpec((BN, 128), lambda i: (i, 0)),
        pl.BlockSpec((BN, 128), lambda i: (i, 0)),
        pl.BlockSpec((1, H), lambda i: (0, 0)),
        pl.BlockSpec((H, H), lambda i: (0, 0)),
    ],
    out_specs=pl.BlockSpec((BN, 128), lambda i: (i, 0)),
    out_shape=jax.ShapeDtypeStruct((NPAD, 128), jnp.float32),
)


def _pool_body(agg_ref, ni_ref, b_ref, gid_ref, sums_ref, counts_ref):
    i = pl.program_id(0)
    aggf = agg_ref[:, :H]
    h = jax.nn.relu(aggf * ni_ref[:, :H] + b_ref[...])
    gidcol = jnp.dot(_expand_grid(gid_ref[0].astype(jnp.float32)),
                     jnp.ones((128, 1), jnp.float32),
                     preferred_element_type=jnp.float32,
                   precision=lax.Precision.HIGHEST)
    cols = lax.broadcasted_iota(jnp.int32, (BN, NG), 1).astype(jnp.float32)
    g = (gidcol == cols).astype(jnp.float32)
    sums = lax.dot_general(g, h, (((0,), (0,)), ((), ())),
                           preferred_element_type=jnp.float32,
                   precision=lax.Precision.HIGHEST)
    counts = lax.dot_general(g, jnp.ones((BN, 1), jnp.float32),
                             (((0,), (0,)), ((), ())),
                             preferred_element_type=jnp.float32,
                   precision=lax.Precision.HIGHEST)

    @pl.when(i == 0)
    def _():
        sums_ref[...] = jnp.zeros_like(sums_ref)
        counts_ref[...] = jnp.zeros_like(counts_ref)

    sums_ref[...] += sums
    counts_ref[...] += counts


_pool_call = pl.pallas_call(
    _pool_body,
    grid=(NT,),
    in_specs=[
        pl.BlockSpec((BN, 128), lambda i: (i, 0)),
        pl.BlockSpec((BN, 128), lambda i: (i, 0)),
        pl.BlockSpec((1, H), lambda i: (0, 0)),
        pl.BlockSpec((1, BN // 128, 128), lambda i: (i, 0, 0)),
    ],
    out_specs=[
        pl.BlockSpec((NG, H), lambda i: (0, 0)),
        pl.BlockSpec((NG, 1), lambda i: (0, 0)),
    ],
    out_shape=[
        jax.ShapeDtypeStruct((NG, H), jnp.float32),
        jax.ShapeDtypeStruct((NG, 1), jnp.float32),
    ],
)


def _combine_body(s1_ref, c1_ref, s2_ref, c2_ref, wc_ref, bc_ref,
                  hg1_ref, hg2_ref, logits_ref):
    hg1 = s1_ref[...] / jnp.maximum(c1_ref[...], 1.0)
    hg2 = s2_ref[...] / jnp.maximum(c2_ref[...], 1.0)
    hg1_ref[...] = hg1
    hg2_ref[...] = hg2
    hg = jnp.abs(hg1 - hg2)
    logits_ref[...] = (
        jnp.dot(hg, wc_ref[...], preferred_element_type=jnp.float32,
                   precision=lax.Precision.HIGHEST)
        + bc_ref[...])


_combine_call = pl.pallas_call(
    _combine_body,
    out_shape=[
        jax.ShapeDtypeStruct((NG, H), jnp.float32),
        jax.ShapeDtypeStruct((NG, H), jnp.float32),
        jax.ShapeDtypeStruct((NG, NC), jnp.float32),
    ],
)


def _pad_edges(v, pad):
    return jnp.concatenate([v, pad]).reshape(ERP, 1, CW)


def _branch_outputs(srcq, dst_r, od, idg, gid, W1, b1, W2, b2, W3, b3, W4, b4):
    od_g = od.reshape(NT, BN // 128, 128)
    id_g = idg.reshape(NT, BN // 128, 128)
    no, ni, m1 = _prep_call(od_g, id_g, W1)
    m = m1
    for b, wn in [(b1, W2), (b2, W3), (b3, W4)]:
        agg, _ = _rowpass_kernel(srcq, dst_r, m)
        m = _layer_call(agg, ni, no, b.reshape(1, H), wn)
    agg4, _ = _rowpass_kernel(srcq, dst_r, m)
    sums, counts = _pool_call(agg4, ni, b4.reshape(1, H), gid)
    return sums, counts


def kernel(edge_index1, edge_index2, graph_ids1, graph_ids2,
           W1, b1, W2, b2, W3, b3, W4, b4, Wc, bc):
    pad = N + (jnp.arange(EPAD, dtype=jnp.int32) % (NPAD - N))
    src1 = _pad_edges(edge_index1[0], pad)
    dst1 = _pad_edges(edge_index1[1], pad)
    src2 = _pad_edges(edge_index2[0], pad)
    dst2 = _pad_edges(edge_index2[1], pad)

    srcs = jnp.stack([src1, src2])
    dsts = jnp.stack([dst1, dst2])

    od, idg = _deg_kernel(srcs, dsts)

    srcq1 = jnp.stack([src1 + q * NPAD for q in range(NQ)])
    srcq2 = jnp.stack([src2 + q * NPAD for q in range(NQ)])

    gpad = jnp.full((NPAD - N,), NG, jnp.int32)
    gid1 = jnp.concatenate([graph_ids1, gpad]).reshape(NT, BN // 128, 128)
    gid2 = jnp.concatenate([graph_ids2, gpad]).reshape(NT, BN // 128, 128)

    s1, c1 = _branch_outputs(srcq1, dst1, od[0], idg[0], gid1,
                             W1, b1, W2, b2, W3, b3, W4, b4)
    s2, c2 = _branch_outputs(srcq2, dst2, od[1], idg[1], gid2,
                             W1, b1, W2, b2, W3, b3, W4, b4)

    return _combine_call(s1, c1, s2, c2, Wc, bc.reshape(1, NC))


# speedup vs baseline: 6.0581x; 1.1644x over previous
import functools

import jax
import jax.numpy as jnp
from jax import lax
from jax.experimental import pallas as pl
from jax.experimental.pallas import tpu as pltpu
from jax.experimental.pallas import tpu_sc as plsc

N = 50000
E = 800000
H = 64
NG = 64
NC = 10

NT = 16
NSC = 2
NQ = 4
QW = H // NQ
NPAD = 51200
BN = NPAD // NT

CW = 128
BLK = 25
NBLK = 16
RPT = BLK * NBLK
ERP = NT * RPT
EPAD = ERP * CW - E

ZR = 400
GW = 13
SR = 400

_mesh = plsc.VectorSubcoreMesh(core_axis_name="c", subcore_axis_name="s")
_sc_params = pltpu.CompilerParams(use_tc_tiling_on_sc=False)


@functools.partial(
    pl.kernel,
    out_type=(
        jax.ShapeDtypeStruct((NSC, NPAD), jnp.float32),
        jax.ShapeDtypeStruct((NSC, NPAD), jnp.float32),
    ),
    mesh=_mesh,
    compiler_params=_sc_params,
    scratch_types=[
        pltpu.VMEM((BLK, 1, CW), jnp.int32),
        pltpu.VMEM((BLK, 1, CW), jnp.int32),
        pltpu.VMEM((CW,), jnp.float32),
        pltpu.VMEM((BN,), jnp.float32),
        pltpu.VMEM_SHARED((NPAD,), jnp.float32),
        pltpu.VMEM_SHARED((NPAD,), jnp.float32),
        pltpu.SemaphoreType.DMA,
    ],
)
def _deg_kernel(srcs_hbm, dsts_hbm, od_hbm, id_hbm,
                src_blk, dst_blk, ones_v, zb, od_sh, id_sh, dsem):
    c = lax.axis_index("c")
    t = lax.axis_index("s")

    one = jnp.full((16,), 1.0, jnp.float32)
    z = jnp.zeros((16,), jnp.float32)
    for i in range(CW // 16):
        ones_v[pl.ds(i * 16, 16)] = one
    for i in range(BN // 16):
        zb[pl.ds(i * 16, 16)] = z

    pltpu.sync_copy(zb, od_sh.at[pl.ds(t * BN, BN)])
    pltpu.sync_copy(zb, id_sh.at[pl.ds(t * BN, BN)])
    plsc.subcore_barrier()

    def block_body(blk, carry):
        row = t * RPT + blk * BLK
        pltpu.sync_copy(srcs_hbm.at[c, pl.ds(row, BLK), :, :], src_blk)
        pltpu.sync_copy(dsts_hbm.at[c, pl.ds(row, BLK), :, :], dst_blk)
        ds_ = []
        for j in range(BLK):
            ds_.append(pltpu.async_copy(
                ones_v, od_sh.at[src_blk.at[j, 0]], dsem, add=True))
            ds_.append(pltpu.async_copy(
                ones_v, id_sh.at[dst_blk.at[j, 0]], dsem, add=True))
        for d in ds_:
            d.wait()
        return carry

    lax.fori_loop(0, NBLK, block_body, 0)
    plsc.subcore_barrier()

    pltpu.sync_copy(od_sh.at[pl.ds(t * BN, BN)], od_hbm.at[c, pl.ds(t * BN, BN)])
    pltpu.sync_copy(id_sh.at[pl.ds(t * BN, BN)], id_hbm.at[c, pl.ds(t * BN, BN)])


@functools.partial(
    pl.kernel,
    out_type=jax.ShapeDtypeStruct((NSC, NPAD), jnp.float32),
    mesh=_mesh,
    compiler_params=_sc_params,
    scratch_types=[
        pltpu.VMEM((BLK, 1, CW), jnp.int32),
        pltpu.VMEM((BLK, 1, CW), jnp.int32),
        pltpu.VMEM((BLK, CW), jnp.float32),
        pltpu.VMEM((BN,), jnp.float32),
        pltpu.VMEM_SHARED((NPAD,), jnp.float32),
        pltpu.SemaphoreType.DMA,
        pltpu.SemaphoreType.DMA,
    ],
)
def _scalar_kernel(srcs_hbm, dsts_hbm, s2x_hbm, sagg_hbm,
                   src_blk, dst_blk, ring, zb, acc_sh, gsem, ssem):
    c = lax.axis_index("c")
    t = lax.axis_index("s")

    z = jnp.zeros((16,), jnp.float32)
    for i in range(BN // 16):
        zb[pl.ds(i * 16, 16)] = z
    pltpu.sync_copy(zb, acc_sh.at[pl.ds(t * BN, BN)])
    plsc.subcore_barrier()

    def block_body(blk, carry):
        row = t * RPT + blk * BLK
        pltpu.sync_copy(srcs_hbm.at[c, pl.ds(row, BLK), :, :], src_blk)
        pltpu.sync_copy(dsts_hbm.at[c, pl.ds(row, BLK), :, :], dst_blk)
        scats = []
        ga = [pltpu.async_copy(s2x_hbm.at[src_blk.at[j, 0]],
                               ring.at[j], gsem)
              for j in range(GW)]
        for j in range(GW):
            ga[j].wait()
        for j in range(GW):
            scats.append(pltpu.async_copy(
                ring.at[j], acc_sh.at[dst_blk.at[j, 0]], ssem, add=True))
        gb = [pltpu.async_copy(s2x_hbm.at[src_blk.at[GW + j, 0]],
                               ring.at[GW + j], gsem)
              for j in range(BLK - GW)]
        for j in range(GW):
            scats[j].wait()
        for j in range(BLK - GW):
            gb[j].wait()
        for j in range(BLK - GW):
            scats.append(pltpu.async_copy(
                ring.at[GW + j], acc_sh.at[dst_blk.at[GW + j, 0]],
                ssem, add=True))
        for d in scats[GW:]:
            d.wait()
        return carry

    lax.fori_loop(0, NBLK, block_body, 0)
    plsc.subcore_barrier()

    pltpu.sync_copy(acc_sh.at[pl.ds(t * BN, BN)],
                    sagg_hbm.at[c, pl.ds(t * BN, BN)])


@functools.partial(
    pl.kernel,
    out_type=(
        jax.ShapeDtypeStruct((NPAD, 128), jnp.float32),
        jax.ShapeDtypeStruct((NQ * NPAD, QW), jnp.float32),
    ),
    mesh=_mesh,
    compiler_params=_sc_params,
    scratch_types=[
        pltpu.VMEM((BLK, 1, CW), jnp.int32),
        pltpu.VMEM((BLK, 1, CW), jnp.int32),
        pltpu.VMEM((BLK, CW, QW), jnp.float32),
        pltpu.VMEM((ZR, QW), jnp.float32),
        pltpu.VMEM_SHARED((NPAD, QW), jnp.float32),
        pltpu.SemaphoreType.DMA,
        pltpu.SemaphoreType.DMA,
    ],
)
def _rowpass_kernel(srcq_hbm, dst_hbm, m_hbm, agg_hbm, msqt_hbm,
                    src_blk, dst_blk, ring, zb, acc_sh,
                    gsem, ssem):
    c = lax.axis_index("c")
    t = lax.axis_index("s")

    z = jnp.zeros((16,), jnp.float32)
    for i in range(ZR):
        zb[i, pl.ds(0, 16)] = z

    for p in range(2):
        q = 2 * c + p
        pltpu.sync_copy(m_hbm.at[pl.ds(t * BN, BN), pl.ds(q * QW, QW)],
                        acc_sh.at[pl.ds(t * BN, BN), :])
        pltpu.sync_copy(acc_sh.at[pl.ds(t * BN, BN), :],
                        msqt_hbm.at[pl.ds(q * NPAD + t * BN, BN), :])

    for p in range(2):
        q = 2 * c + p

        for k in range(BN // ZR):
            pltpu.sync_copy(zb, acc_sh.at[pl.ds(t * BN + k * ZR, ZR), :])
        plsc.subcore_barrier()

        def block_body(blk, carry):
            row = t * RPT + blk * BLK
            pltpu.sync_copy(srcq_hbm.at[q, pl.ds(row, BLK), :, :], src_blk)
            pltpu.sync_copy(dst_hbm.at[pl.ds(row, BLK), :, :], dst_blk)
            scats = []
            ga = [pltpu.async_copy(msqt_hbm.at[src_blk.at[j, 0]],
                                   ring.at[j], gsem)
                  for j in range(GW)]
            for j in range(GW):
                ga[j].wait()
            for j in range(GW):
                scats.append(pltpu.async_copy(
                    ring.at[j], acc_sh.at[dst_blk.at[j, 0]], ssem, add=True))
            gb = [pltpu.async_copy(msqt_hbm.at[src_blk.at[GW + j, 0]],
                                   ring.at[GW + j], gsem)
                  for j in range(BLK - GW)]
            for j in range(GW):
                scats[j].wait()
            for j in range(BLK - GW):
                gb[j].wait()
            for j in range(BLK - GW):
                scats.append(pltpu.async_copy(
                    ring.at[GW + j], acc_sh.at[dst_blk.at[GW + j, 0]],
                    ssem, add=True))
            for d in scats[GW:]:
                d.wait()
            return carry

        lax.fori_loop(0, NBLK, block_body, 0)
        plsc.subcore_barrier()

        pltpu.sync_copy(acc_sh.at[pl.ds(t * BN, BN), :],
                        agg_hbm.at[pl.ds(t * BN, BN), pl.ds(q * QW, QW)])


def _expand_grid(grid_vals):
    gb = grid_vals.shape[0]
    rep = jnp.broadcast_to(grid_vals[:, None, :], (gb, 128, 128))
    rep = rep.reshape(gb * 128, 128)
    lane = lax.broadcasted_iota(jnp.int32, (gb * 128, 128), 1)
    rowmod = lax.broadcasted_iota(jnp.int32, (gb * 128, 128), 0) % 128
    return jnp.where(lane == rowmod, rep, 0.0)


def _to_dense(grid_vals, k):
    a = _expand_grid(grid_vals)
    return jnp.dot(a, jnp.ones((128, k), jnp.float32),
                   preferred_element_type=jnp.float32,
                   precision=lax.Precision.HIGHEST)


def _prep_body(od_ref, id_ref, nino_ref, s_ref):
    no_g = lax.rsqrt(jnp.maximum(od_ref[0], 1.0))
    ni_g = lax.rsqrt(jnp.maximum(id_ref[0], 1.0))
    nino_ref[:, :H] = _to_dense(ni_g, H)
    nino_ref[:, H:] = _to_dense(no_g, H)
    s_ref[0, :, :] = id_ref[0] * no_g


_prep_call = pl.pallas_call(
    _prep_body,
    grid=(NT,),
    in_specs=[
        pl.BlockSpec((1, BN // 128, 128), lambda i: (i, 0, 0)),
        pl.BlockSpec((1, BN // 128, 128), lambda i: (i, 0, 0)),
    ],
    out_specs=[
        pl.BlockSpec((BN, 128), lambda i: (i, 0)),
        pl.BlockSpec((1, BN // 128, 128), lambda i: (i, 0, 0)),
    ],
    out_shape=[
        jax.ShapeDtypeStruct((NPAD, 128), jnp.float32),
        jax.ShapeDtypeStruct((NT, BN // 128, 128), jnp.float32),
    ],
)


def _layer1_body(od_ref, id_ref, sagg_ref, w1_ref, b1_ref, w2_ref, m_ref):
    no_g = lax.rsqrt(jnp.maximum(od_ref[0], 1.0))
    ni_g = lax.rsqrt(jnp.maximum(id_ref[0], 1.0))
    w1b = jnp.broadcast_to(w1_ref[...], (128, H))
    h1 = jax.nn.relu(
        jnp.dot(_expand_grid(sagg_ref[0] * ni_g), w1b,
                preferred_element_type=jnp.float32,
                precision=lax.Precision.HIGHEST) + b1_ref[...])
    no_d = _to_dense(no_g, H)
    m = jnp.dot(h1 * no_d, w2_ref[...], preferred_element_type=jnp.float32,
                precision=lax.Precision.HIGHEST)
    m_ref[:, :H] = m
    m_ref[:, H:] = jnp.zeros((BN, 128 - H), jnp.float32)


_layer1_call = pl.pallas_call(
    _layer1_body,
    grid=(NT,),
    in_specs=[
        pl.BlockSpec((1, BN // 128, 128), lambda i: (i, 0, 0)),
        pl.BlockSpec((1, BN // 128, 128), lambda i: (i, 0, 0)),
        pl.BlockSpec((1, BN // 128, 128), lambda i: (i, 0, 0)),
        pl.BlockSpec((1, H), lambda i: (0, 0)),
        pl.BlockSpec((1, H), lambda i: (0, 0)),
        pl.BlockSpec((H, H), lambda i: (0, 0)),
    ],
    out_specs=pl.BlockSpec((BN, 128), lambda i: (i, 0)),
    out_shape=jax.ShapeDtypeStruct((NPAD, 128), jnp.float32),
)


def _layer_body(agg_ref, nino_ref, b_ref, w_ref, m_ref):
    h = jax.nn.relu(agg_ref[:, :H] * nino_ref[:, :H] + b_ref[...])
    m = jnp.dot(h * nino_ref[:, H:], w_ref[...],
                preferred_element_type=jnp.float32,
                precision=lax.Precision.HIGHEST)
    m_ref[:, :H] = m
    m_ref[:, H:] = jnp.zeros((BN, 128 - H), jnp.float32)


_layer_call = pl.pallas_call(
    _layer_body,
    grid=(NT,),
    in_specs=[
        pl.BlockSpec((BN, 128), lambda i: (i, 0)),
        pl.BlockSpec((BN, 128), lambda i: (i, 0)),
        pl.BlockSpec((1, H), lambda i: (0, 0)),
        pl.BlockSpec((H, H), lambda i: (0, 0)),
    ],
    out_specs=pl.BlockSpec((BN, 128), lambda i: (i, 0)),
    out_shape=jax.ShapeDtypeStruct((NPAD, 128), jnp.float32),
)


def _pool_body(agg_ref, ni_ref, b_ref, gid_ref, sums_ref, counts_ref):
    i = pl.program_id(0)
    aggf = agg_ref[:, :H]
    h = jax.nn.relu(aggf * ni_ref[:, :H] + b_ref[...])
    gidcol = jnp.dot(_expand_grid(gid_ref[0].astype(jnp.float32)),
                     jnp.ones((128, 1), jnp.float32),
                     preferred_element_type=jnp.float32,
                   precision=lax.Precision.HIGHEST)
    cols = lax.broadcasted_iota(jnp.int32, (BN, NG), 1).astype(jnp.float32)
    g = (gidcol == cols).astype(jnp.float32)
    sums = lax.dot_general(g, h, (((0,), (0,)), ((), ())),
                           preferred_element_type=jnp.float32,
                   precision=lax.Precision.HIGHEST)
    counts = lax.dot_general(g, jnp.ones((BN, 1), jnp.float32),
                             (((0,), (0,)), ((), ())),
                             preferred_element_type=jnp.float32,
                   precision=lax.Precision.HIGHEST)

    @pl.when(i == 0)
    def _():
        sums_ref[...] = jnp.zeros_like(sums_ref)
        counts_ref[...] = jnp.zeros_like(counts_ref)

    sums_ref[...] += sums
    counts_ref[...] += counts


_pool_call = pl.pallas_call(
    _pool_body,
    grid=(NT,),
    in_specs=[
        pl.BlockSpec((BN, 128), lambda i: (i, 0)),
        pl.BlockSpec((BN, 128), lambda i: (i, 0)),
        pl.BlockSpec((1, H), lambda i: (0, 0)),
        pl.BlockSpec((1, BN // 128, 128), lambda i: (i, 0, 0)),
    ],
    out_specs=[
        pl.BlockSpec((NG, H), lambda i: (0, 0)),
        pl.BlockSpec((NG, 1), lambda i: (0, 0)),
    ],
    out_shape=[
        jax.ShapeDtypeStruct((NG, H), jnp.float32),
        jax.ShapeDtypeStruct((NG, 1), jnp.float32),
    ],
)


def _combine_body(s1_ref, c1_ref, s2_ref, c2_ref, wc_ref, bc_ref,
                  hg1_ref, hg2_ref, logits_ref):
    hg1 = s1_ref[...] / jnp.maximum(c1_ref[...], 1.0)
    hg2 = s2_ref[...] / jnp.maximum(c2_ref[...], 1.0)
    hg1_ref[...] = hg1
    hg2_ref[...] = hg2
    hg = jnp.abs(hg1 - hg2)
    logits_ref[...] = (
        jnp.dot(hg, wc_ref[...], preferred_element_type=jnp.float32,
                   precision=lax.Precision.HIGHEST)
        + bc_ref[...])


_combine_call = pl.pallas_call(
    _combine_body,
    out_shape=[
        jax.ShapeDtypeStruct((NG, H), jnp.float32),
        jax.ShapeDtypeStruct((NG, H), jnp.float32),
        jax.ShapeDtypeStruct((NG, NC), jnp.float32),
    ],
)


def _pad_edges(v, pad):
    return jnp.concatenate([v, pad]).reshape(ERP, 1, CW)


def _branch_outputs(srcq, dst_r, od_g, id_g, sagg_g, nino, gid,
                    W1, b1, W2, b2, W3, b3, W4, b4):
    m = _layer1_call(od_g, id_g, sagg_g, W1, b1.reshape(1, H), W2)
    for b, wn in [(b2, W3), (b3, W4)]:
        agg, _ = _rowpass_kernel(srcq, dst_r, m)
        m = _layer_call(agg, nino, b.reshape(1, H), wn)
    agg4, _ = _rowpass_kernel(srcq, dst_r, m)
    sums, counts = _pool_call(agg4, nino, b4.reshape(1, H), gid)
    return sums, counts


def kernel(edge_index1, edge_index2, graph_ids1, graph_ids2,
           W1, b1, W2, b2, W3, b3, W4, b4, Wc, bc):
    pad = N + (jnp.arange(EPAD, dtype=jnp.int32) % (NPAD - N))
    src1 = _pad_edges(edge_index1[0], pad)
    dst1 = _pad_edges(edge_index1[1], pad)
    src2 = _pad_edges(edge_index2[0], pad)
    dst2 = _pad_edges(edge_index2[1], pad)

    srcs = jnp.stack([src1, src2])
    dsts = jnp.stack([dst1, dst2])

    od, idg = _deg_kernel(srcs, dsts)

    od_g1 = od[0].reshape(NT, BN // 128, 128)
    id_g1 = idg[0].reshape(NT, BN // 128, 128)
    od_g2 = od[1].reshape(NT, BN // 128, 128)
    id_g2 = idg[1].reshape(NT, BN // 128, 128)

    nino1, s_g1 = _prep_call(od_g1, id_g1)
    nino2, s_g2 = _prep_call(od_g2, id_g2)

    s2x = jnp.stack([s_g1, s_g2]).reshape(2 * NPAD)
    srcs_off = srcs + jnp.array([0, NPAD], jnp.int32).reshape(2, 1, 1, 1)
    sagg = _scalar_kernel(srcs_off, dsts, s2x)
    sagg_g1 = sagg[0].reshape(NT, BN // 128, 128)
    sagg_g2 = sagg[1].reshape(NT, BN // 128, 128)

    srcq1 = jnp.stack([src1 + q * NPAD for q in range(NQ)])
    srcq2 = jnp.stack([src2 + q * NPAD for q in range(NQ)])

    gpad = jnp.full((NPAD - N,), NG, jnp.int32)
    gid1 = jnp.concatenate([graph_ids1, gpad]).reshape(NT, BN // 128, 128)
    gid2 = jnp.concatenate([graph_ids2, gpad]).reshape(NT, BN // 128, 128)

    s1, c1 = _branch_outputs(srcq1, dst1, od_g1, id_g1, sagg_g1, nino1, gid1,
                             W1, b1, W2, b2, W3, b3, W4, b4)
    s2, c2 = _branch_outputs(srcq2, dst2, od_g2, id_g2, sagg_g2, nino2, gid2,
                             W1, b1, W2, b2, W3, b3, W4, b4)

    return _combine_call(s1, c1, s2, c2, Wc, bc.reshape(1, NC))


# speedup vs baseline: 6.1686x; 1.0182x over previous
import functools

import jax
import jax.numpy as jnp
from jax import lax
from jax.experimental import pallas as pl
from jax.experimental.pallas import tpu as pltpu
from jax.experimental.pallas import tpu_sc as plsc

N = 50000
E = 800000
H = 64
NG = 64
NC = 10

NT = 16
NSC = 2
NQ = 4
QW = H // NQ
NPAD = 51200
BN = NPAD // NT

CW = 128
BLK = 25
NBLK = 16
RPT = BLK * NBLK
ERP = NT * RPT
EPAD = ERP * CW - E

ZR = 400
GW = 13
SR = 400

_mesh = plsc.VectorSubcoreMesh(core_axis_name="c", subcore_axis_name="s")
_sc_params = pltpu.CompilerParams(use_tc_tiling_on_sc=False)


@functools.partial(
    pl.kernel,
    out_type=(
        jax.ShapeDtypeStruct((NSC, NPAD), jnp.float32),
        jax.ShapeDtypeStruct((NSC, NPAD), jnp.float32),
    ),
    mesh=_mesh,
    compiler_params=_sc_params,
    scratch_types=[
        pltpu.VMEM((BLK, 1, CW), jnp.int32),
        pltpu.VMEM((BLK, 1, CW), jnp.int32),
        pltpu.VMEM((CW,), jnp.float32),
        pltpu.VMEM((BN,), jnp.float32),
        pltpu.VMEM_SHARED((NPAD,), jnp.float32),
        pltpu.VMEM_SHARED((NPAD,), jnp.float32),
        pltpu.SemaphoreType.DMA,
    ],
)
def _deg_kernel(srcs_hbm, dsts_hbm, od_hbm, id_hbm,
                src_blk, dst_blk, ones_v, zb, od_sh, id_sh, dsem):
    c = lax.axis_index("c")
    t = lax.axis_index("s")

    one = jnp.full((16,), 1.0, jnp.float32)
    z = jnp.zeros((16,), jnp.float32)
    for i in range(CW // 16):
        ones_v[pl.ds(i * 16, 16)] = one
    for i in range(BN // 16):
        zb[pl.ds(i * 16, 16)] = z

    pltpu.sync_copy(zb, od_sh.at[pl.ds(t * BN, BN)])
    pltpu.sync_copy(zb, id_sh.at[pl.ds(t * BN, BN)])
    plsc.subcore_barrier()

    def block_body(blk, carry):
        row = t * RPT + blk * BLK
        pltpu.sync_copy(srcs_hbm.at[c, pl.ds(row, BLK), :, :], src_blk)
        pltpu.sync_copy(dsts_hbm.at[c, pl.ds(row, BLK), :, :], dst_blk)
        ds_ = []
        for j in range(BLK):
            ds_.append(pltpu.async_copy(
                ones_v, od_sh.at[src_blk.at[j, 0]], dsem, add=True))
            ds_.append(pltpu.async_copy(
                ones_v, id_sh.at[dst_blk.at[j, 0]], dsem, add=True))
        for d in ds_:
            d.wait()
        return carry

    lax.fori_loop(0, NBLK, block_body, 0)
    plsc.subcore_barrier()

    pltpu.sync_copy(od_sh.at[pl.ds(t * BN, BN)], od_hbm.at[c, pl.ds(t * BN, BN)])
    pltpu.sync_copy(id_sh.at[pl.ds(t * BN, BN)], id_hbm.at[c, pl.ds(t * BN, BN)])


@functools.partial(
    pl.kernel,
    out_type=jax.ShapeDtypeStruct((NSC, NPAD), jnp.float32),
    mesh=_mesh,
    compiler_params=_sc_params,
    scratch_types=[
        pltpu.VMEM((BLK, 1, CW), jnp.int32),
        pltpu.VMEM((BLK, 1, CW), jnp.int32),
        pltpu.VMEM((BLK, CW), jnp.float32),
        pltpu.VMEM((BN,), jnp.float32),
        pltpu.VMEM_SHARED((NPAD,), jnp.float32),
        pltpu.SemaphoreType.DMA,
        pltpu.SemaphoreType.DMA,
    ],
)
def _scalar_kernel(srcs_hbm, dsts_hbm, s2x_hbm, sagg_hbm,
                   src_blk, dst_blk, ring, zb, acc_sh, gsem, ssem):
    c = lax.axis_index("c")
    t = lax.axis_index("s")

    z = jnp.zeros((16,), jnp.float32)
    for i in range(BN // 16):
        zb[pl.ds(i * 16, 16)] = z
    pltpu.sync_copy(zb, acc_sh.at[pl.ds(t * BN, BN)])
    plsc.subcore_barrier()

    def block_body(blk, carry):
        row = t * RPT + blk * BLK
        pltpu.sync_copy(srcs_hbm.at[c, pl.ds(row, BLK), :, :], src_blk)
        pltpu.sync_copy(dsts_hbm.at[c, pl.ds(row, BLK), :, :], dst_blk)
        scats = []
        ga = [pltpu.async_copy(s2x_hbm.at[src_blk.at[j, 0]],
                               ring.at[j], gsem)
              for j in range(GW)]
        for j in range(GW):
            ga[j].wait()
        for j in range(GW):
            scats.append(pltpu.async_copy(
                ring.at[j], acc_sh.at[dst_blk.at[j, 0]], ssem, add=True))
        gb = [pltpu.async_copy(s2x_hbm.at[src_blk.at[GW + j, 0]],
                               ring.at[GW + j], gsem)
              for j in range(BLK - GW)]
        for j in range(GW):
            scats[j].wait()
        for j in range(BLK - GW):
            gb[j].wait()
        for j in range(BLK - GW):
            scats.append(pltpu.async_copy(
                ring.at[GW + j], acc_sh.at[dst_blk.at[GW + j, 0]],
                ssem, add=True))
        for d in scats[GW:]:
            d.wait()
        return carry

    lax.fori_loop(0, NBLK, block_body, 0)
    plsc.subcore_barrier()

    pltpu.sync_copy(acc_sh.at[pl.ds(t * BN, BN)],
                    sagg_hbm.at[c, pl.ds(t * BN, BN)])


@functools.partial(
    pl.kernel,
    out_type=(
        jax.ShapeDtypeStruct((NPAD, 128), jnp.float32),
        jax.ShapeDtypeStruct((NQ * NPAD, QW), jnp.float32),
    ),
    mesh=_mesh,
    compiler_params=_sc_params,
    scratch_types=[
        pltpu.VMEM((BLK, 1, CW), jnp.int32),
        pltpu.VMEM((BLK, 1, CW), jnp.int32),
        pltpu.VMEM((BLK, CW, QW), jnp.float32),
        pltpu.VMEM((ZR, QW), jnp.float32),
        pltpu.VMEM_SHARED((NPAD, QW), jnp.float32),
        pltpu.SemaphoreType.DMA,
        pltpu.SemaphoreType.DMA,
    ],
)
def _rowpass_kernel(srcq_hbm, dst_hbm, m_hbm, agg_hbm, msqt_hbm,
                    src_blk, dst_blk, ring, zb, acc_sh,
                    gsem, ssem):
    c = lax.axis_index("c")
    t = lax.axis_index("s")

    z = jnp.zeros((16,), jnp.float32)
    for i in range(ZR):
        zb[i, pl.ds(0, 16)] = z

    for p in range(2):
        q = 2 * c + p
        pltpu.sync_copy(m_hbm.at[pl.ds(t * BN, BN), pl.ds(q * QW, QW)],
                        acc_sh.at[pl.ds(t * BN, BN), :])
        pltpu.sync_copy(acc_sh.at[pl.ds(t * BN, BN), :],
                        msqt_hbm.at[pl.ds(q * NPAD + t * BN, BN), :])

    for p in range(2):
        q = 2 * c + p

        for k in range(BN // ZR):
            pltpu.sync_copy(zb, acc_sh.at[pl.ds(t * BN + k * ZR, ZR), :])
        plsc.subcore_barrier()

        def block_body(blk, carry):
            row = t * RPT + blk * BLK
            pltpu.sync_copy(srcq_hbm.at[q, pl.ds(row, BLK), :, :], src_blk)
            pltpu.sync_copy(dst_hbm.at[pl.ds(row, BLK), :, :], dst_blk)
            scats = []
            ga = [pltpu.async_copy(msqt_hbm.at[src_blk.at[j, 0]],
                                   ring.at[j], gsem)
                  for j in range(GW)]
            for j in range(GW):
                ga[j].wait()
            for j in range(GW):
                scats.append(pltpu.async_copy(
                    ring.at[j], acc_sh.at[dst_blk.at[j, 0]], ssem, add=True))
            gb = [pltpu.async_copy(msqt_hbm.at[src_blk.at[GW + j, 0]],
                                   ring.at[GW + j], gsem)
                  for j in range(BLK - GW)]
            for j in range(GW):
                scats[j].wait()
            for j in range(BLK - GW):
                gb[j].wait()
            for j in range(BLK - GW):
                scats.append(pltpu.async_copy(
                    ring.at[GW + j], acc_sh.at[dst_blk.at[GW + j, 0]],
                    ssem, add=True))
            for d in scats[GW:]:
                d.wait()
            return carry

        lax.fori_loop(0, NBLK, block_body, 0)
        plsc.subcore_barrier()

        pltpu.sync_copy(acc_sh.at[pl.ds(t * BN, BN), :],
                        agg_hbm.at[pl.ds(t * BN, BN), pl.ds(q * QW, QW)])


def _expand_grid(grid_vals):
    gb = grid_vals.shape[0]
    rep = jnp.broadcast_to(grid_vals[:, None, :], (gb, 128, 128))
    rep = rep.reshape(gb * 128, 128)
    lane = lax.broadcasted_iota(jnp.int32, (gb * 128, 128), 1)
    rowmod = lax.broadcasted_iota(jnp.int32, (gb * 128, 128), 0) % 128
    return jnp.where(lane == rowmod, rep, 0.0)


def _to_dense(grid_vals, k):
    a = _expand_grid(grid_vals)
    return jnp.dot(a, jnp.ones((128, k), jnp.float32),
                   preferred_element_type=jnp.float32,
                   precision=lax.Precision.HIGHEST)


def _prep_body(od_ref, id_ref, nino_ref, s_ref):
    no_g = lax.rsqrt(jnp.maximum(od_ref[0], 1.0))
    ni_g = lax.rsqrt(jnp.maximum(id_ref[0], 1.0))
    nino_ref[:, :H] = _to_dense(ni_g, H)
    nino_ref[:, H:] = _to_dense(no_g, H)
    s_ref[0, :, :] = id_ref[0] * no_g


_prep_call = pl.pallas_call(
    _prep_body,
    grid=(NT,),
    in_specs=[
        pl.BlockSpec((1, BN // 128, 128), lambda i: (i, 0, 0)),
        pl.BlockSpec((1, BN // 128, 128), lambda i: (i, 0, 0)),
    ],
    out_specs=[
        pl.BlockSpec((BN, 128), lambda i: (i, 0)),
        pl.BlockSpec((1, BN // 128, 128), lambda i: (i, 0, 0)),
    ],
    out_shape=[
        jax.ShapeDtypeStruct((NPAD, 128), jnp.float32),
        jax.ShapeDtypeStruct((NT, BN // 128, 128), jnp.float32),
    ],
)


def _layer1_body(od_ref, id_ref, sagg_ref, w1_ref, b1_ref, w2_ref, m_ref):
    no_g = lax.rsqrt(jnp.maximum(od_ref[0], 1.0))
    ni_g = lax.rsqrt(jnp.maximum(id_ref[0], 1.0))
    w1b = jnp.broadcast_to(w1_ref[...], (128, H))
    h1 = jax.nn.relu(
        jnp.dot(_expand_grid(sagg_ref[0] * ni_g), w1b,
                preferred_element_type=jnp.float32,
                precision=lax.Precision.HIGHEST) + b1_ref[...])
    no_d = _to_dense(no_g, H)
    m = jnp.dot(h1 * no_d, w2_ref[...], preferred_element_type=jnp.float32)
    m_ref[:, :H] = m
    m_ref[:, H:] = jnp.zeros((BN, 128 - H), jnp.float32)


_layer1_call = pl.pallas_call(
    _layer1_body,
    grid=(NT,),
    in_specs=[
        pl.BlockSpec((1, BN // 128, 128), lambda i: (i, 0, 0)),
        pl.BlockSpec((1, BN // 128, 128), lambda i: (i, 0, 0)),
        pl.BlockSpec((1, BN // 128, 128), lambda i: (i, 0, 0)),
        pl.BlockSpec((1, H), lambda i: (0, 0)),
        pl.BlockSpec((1, H), lambda i: (0, 0)),
        pl.BlockSpec((H, H), lambda i: (0, 0)),
    ],
    out_specs=pl.BlockSpec((BN, 128), lambda i: (i, 0)),
    out_shape=jax.ShapeDtypeStruct((NPAD, 128), jnp.float32),
)


def _layer_body(agg_ref, nino_ref, b_ref, w_ref, m_ref):
    h = jax.nn.relu(agg_ref[:, :H] * nino_ref[:, :H] + b_ref[...])
    m = jnp.dot(h * nino_ref[:, H:], w_ref[...],
                preferred_element_type=jnp.float32)
    m_ref[:, :H] = m
    m_ref[:, H:] = jnp.zeros((BN, 128 - H), jnp.float32)


_layer_call = pl.pallas_call(
    _layer_body,
    grid=(NT,),
    in_specs=[
        pl.BlockSpec((BN, 128), lambda i: (i, 0)),
        pl.BlockSpec((BN, 128), lambda i: (i, 0)),
        pl.BlockSpec((1, H), lambda i: (0, 0)),
        pl.BlockSpec((H, H), lambda i: (0, 0)),
    ],
    out_specs=pl.BlockSpec((BN, 128), lambda i: (i, 0)),
    out_shape=jax.ShapeDtypeStruct((NPAD, 128), jnp.float32),
)


def _pool_body(agg_ref, ni_ref, b_ref, gid_ref, sums_ref, counts_ref):
    i = pl.program_id(0)
    aggf = agg_ref[:, :H]
    h = jax.nn.relu(aggf * ni_ref[:, :H] + b_ref[...])
    gidcol = jnp.dot(_expand_grid(gid_ref[0].astype(jnp.float32)),
                     jnp.ones((128, 1), jnp.float32),
                     preferred_element_type=jnp.float32,
                   precision=lax.Precision.HIGHEST)
    cols = lax.broadcasted_iota(jnp.int32, (BN, NG), 1).astype(jnp.float32)
    g = (gidcol == cols).astype(jnp.float32)
    sums = lax.dot_general(g, h, (((0,), (0,)), ((), ())),
                           preferred_element_type=jnp.float32,
                   precision=lax.Precision.HIGHEST)
    counts = lax.dot_general(g, jnp.ones((BN, 1), jnp.float32),
                             (((0,), (0,)), ((), ())),
                             preferred_element_type=jnp.float32,
                   precision=lax.Precision.HIGHEST)

    @pl.when(i == 0)
    def _():
        sums_ref[...] = jnp.zeros_like(sums_ref)
        counts_ref[...] = jnp.zeros_like(counts_ref)

    sums_ref[...] += sums
    counts_ref[...] += counts


_pool_call = pl.pallas_call(
    _pool_body,
    grid=(NT,),
    in_specs=[
        pl.BlockSpec((BN, 128), lambda i: (i, 0)),
        pl.BlockSpec((BN, 128), lambda i: (i, 0)),
        pl.BlockSpec((1, H), lambda i: (0, 0)),
        pl.BlockSpec((1, BN // 128, 128), lambda i: (i, 0, 0)),
    ],
    out_specs=[
        pl.BlockSpec((NG, H), lambda i: (0, 0)),
        pl.BlockSpec((NG, 1), lambda i: (0, 0)),
    ],
    out_shape=[
        jax.ShapeDtypeStruct((NG, H), jnp.float32),
        jax.ShapeDtypeStruct((NG, 1), jnp.float32),
    ],
)


def _combine_body(s1_ref, c1_ref, s2_ref, c2_ref, wc_ref, bc_ref,
                  hg1_ref, hg2_ref, logits_ref):
    hg1 = s1_ref[...] / jnp.maximum(c1_ref[...], 1.0)
    hg2 = s2_ref[...] / jnp.maximum(c2_ref[...], 1.0)
    hg1_ref[...] = hg1
    hg2_ref[...] = hg2
    hg = jnp.abs(hg1 - hg2)
    logits_ref[...] = (
        jnp.dot(hg, wc_ref[...], preferred_element_type=jnp.float32,
                   precision=lax.Precision.HIGHEST)
        + bc_ref[...])


_combine_call = pl.pallas_call(
    _combine_body,
    out_shape=[
        jax.ShapeDtypeStruct((NG, H), jnp.float32),
        jax.ShapeDtypeStruct((NG, H), jnp.float32),
        jax.ShapeDtypeStruct((NG, NC), jnp.float32),
    ],
)


def _pad_edges(v, pad):
    return jnp.concatenate([v, pad]).reshape(ERP, 1, CW)


def _branch_outputs(srcq, dst_r, od_g, id_g, sagg_g, nino, gid,
                    W1, b1, W2, b2, W3, b3, W4, b4):
    m = _layer1_call(od_g, id_g, sagg_g, W1, b1.reshape(1, H), W2)
    for b, wn in [(b2, W3), (b3, W4)]:
        agg, _ = _rowpass_kernel(srcq, dst_r, m)
        m = _layer_call(agg, nino, b.reshape(1, H), wn)
    agg4, _ = _rowpass_kernel(srcq, dst_r, m)
    sums, counts = _pool_call(agg4, nino, b4.reshape(1, H), gid)
    return sums, counts


def kernel(edge_index1, edge_index2, graph_ids1, graph_ids2,
           W1, b1, W2, b2, W3, b3, W4, b4, Wc, bc):
    pad = N + (jnp.arange(EPAD, dtype=jnp.int32) % (NPAD - N))
    src1 = _pad_edges(edge_index1[0], pad)
    dst1 = _pad_edges(edge_index1[1], pad)
    src2 = _pad_edges(edge_index2[0], pad)
    dst2 = _pad_edges(edge_index2[1], pad)

    srcs = jnp.stack([src1, src2])
    dsts = jnp.stack([dst1, dst2])

    od, idg = _deg_kernel(srcs, dsts)

    od_g1 = od[0].reshape(NT, BN // 128, 128)
    id_g1 = idg[0].reshape(NT, BN // 128, 128)
    od_g2 = od[1].reshape(NT, BN // 128, 128)
    id_g2 = idg[1].reshape(NT, BN // 128, 128)

    nino1, s_g1 = _prep_call(od_g1, id_g1)
    nino2, s_g2 = _prep_call(od_g2, id_g2)

    s2x = jnp.stack([s_g1, s_g2]).reshape(2 * NPAD)
    srcs_off = srcs + jnp.array([0, NPAD], jnp.int32).reshape(2, 1, 1, 1)
    sagg = _scalar_kernel(srcs_off, dsts, s2x)
    sagg_g1 = sagg[0].reshape(NT, BN // 128, 128)
    sagg_g2 = sagg[1].reshape(NT, BN // 128, 128)

    srcq1 = jnp.stack([src1 + q * NPAD for q in range(NQ)])
    srcq2 = jnp.stack([src2 + q * NPAD for q in range(NQ)])

    gpad = jnp.full((NPAD - N,), NG, jnp.int32)
    gid1 = jnp.concatenate([graph_ids1, gpad]).reshape(NT, BN // 128, 128)
    gid2 = jnp.concatenate([graph_ids2, gpad]).reshape(NT, BN // 128, 128)

    s1, c1 = _branch_outputs(srcq1, dst1, od_g1, id_g1, sagg_g1, nino1, gid1,
                             W1, b1, W2, b2, W3, b3, W4, b4)
    s2, c2 = _branch_outputs(srcq2, dst2, od_g2, id_g2, sagg_g2, nino2, gid2,
                             W1, b1, W2, b2, W3, b3, W4, b4)

    return _combine_call(s1, c1, s2, c2, Wc, bc.reshape(1, NC))
